# baseline (speedup 1.0000x reference)
import functools

import jax
import jax.numpy as jnp
from jax import lax
from jax.experimental import pallas as pl
from jax.experimental.pallas import tpu as pltpu
from jax.experimental.pallas import tpu_sc as plsc

B, L, D, S, K, H, NB = 2, 2048, 1024, 16, 4, 16, 2
DH = D // H
HS = H * S
TL = 512
EPS = 1e-6
NEG = -3.0e38


def _rms(t, g):
    return t * lax.rsqrt(jnp.mean(t * t, axis=-1, keepdims=True) + EPS) * g[None, :]


def _a_body(x_ref, Wc_ref, bc_ref, gpre_ref, lt_ref):
    for b in range(B):
        xn = _rms(x_ref[b], gpre_ref[...])
        lgT = lax.dot_general(Wc_ref[...], xn, (((1,), (1,)), ((), ())),
                              preferred_element_type=jnp.float32)
        lt_ref[b] = lgT + bc_ref[...][:, None]


def _call_a(x, Wc, bc, g_pre):
    return pl.pallas_call(
        _a_body,
        grid=(L // TL,),
        in_specs=[
            pl.BlockSpec((B, TL, D), lambda t: (0, t, 0)),
            pl.BlockSpec((S, D), lambda t: (0, 0)),
            pl.BlockSpec((S,), lambda t: (0,)),
            pl.BlockSpec((D,), lambda t: (0,)),
        ],
        out_specs=pl.BlockSpec((B, S, TL), lambda t: (0, 0, t)),
        out_shape=jax.ShapeDtypeStruct((B, S, L), jnp.float32),
    )(x, Wc, bc, g_pre)


def _b_body(lt_hbm, x2d_hbm, ti_hbm, tv_hbm, rows_hbm,
            lg_v, iv_v, tv_v, rows_v, sem):
    wid = lax.axis_index("c") * 16 + lax.axis_index("s")
    b = wid // S
    s = wid % S
    pltpu.sync_copy(lt_hbm.at[b, s], lg_v)
    lane = lax.iota(jnp.int32, 16)

    UNROLL = 8
    selv = []
    seli = []
    for _ in range(K):
        prev = list(seli)

        def scan_body(i, carry, _prev=prev):
            bv, bi_ = carry
            cand = []
            for u in range(UNROLL):
                off = pl.multiple_of((i * UNROLL + u) * 16, 16)
                v = lg_v[pl.ds(off, 16)]
                idx = off + lane
                for pj in _prev:
                    v = jnp.where(idx == pj, NEG, v)
                cand.append((v, idx))
            while len(cand) > 1:
                nxt = []
                for j in range(0, len(cand), 2):
                    (va, ia), (vb, ib) = cand[j], cand[j + 1]
                    keep_a = va >= vb
                    nxt.append((jnp.where(keep_a, va, vb),
                                jnp.where(keep_a, ia, ib)))
                cand = nxt
            v, idx = cand[0]
            upd = v > bv
            return jnp.where(upd, v, bv), jnp.where(upd, idx, bi_)

        bv, bi_ = lax.fori_loop(
            0, L // (16 * UNROLL), scan_body,
            (jnp.full((16,), NEG, jnp.float32), jnp.zeros((16,), jnp.int32)))
        m = jnp.max(bv)
        cand = jnp.where(bv == m, bi_, jnp.int32(2 ** 30))
        selv.append(m)
        seli.append(jnp.min(cand))

    for a, c in ((0, 1), (2, 3), (0, 2), (1, 3), (1, 2)):
        swap = seli[a] > seli[c]
        ia, ic = seli[a], seli[c]
        va, vc = selv[a], selv[c]
        seli[a] = jnp.where(swap, ic, ia)
        seli[c] = jnp.where(swap, ia, ic)
        selv[a] = jnp.where(swap, vc, va)
        selv[c] = jnp.where(swap, va, vc)

    vvec = jnp.zeros((16,), jnp.float32)
    ivec = jnp.zeros((16,), jnp.int32)
    for j in range(K):
        vvec = jnp.where(lane == j, selv[j], vvec)
        ivec = jnp.where(lane == j, seli[j], ivec)
    iv_v[...] = ivec
    tv_v[...] = vvec
    pltpu.sync_copy(iv_v, ti_hbm.at[b, s])
    pltpu.sync_copy(tv_v, tv_hbm.at[b, s])

    gidx = ivec + b * L
    pltpu.async_copy(x2d_hbm.at[gidx], rows_v, sem).wait()
    pltpu.sync_copy(rows_v.at[pl.ds(0, K)], rows_hbm.at[b, s])


def _sc_topk(ltT, x2d):
    mesh = plsc.VectorSubcoreMesh(core_axis_name="c", subcore_axis_name="s")
    return pl.kernel(
        _b_body,
        out_type=[jax.ShapeDtypeStruct((B, S, 16), jnp.int32),
                  jax.ShapeDtypeStruct((B, S, 16), jnp.float32),
                  jax.ShapeDtypeStruct((B, S, K, D), jnp.float32)],
        mesh=mesh,
        compiler_params=pltpu.CompilerParams(needs_layout_passes=False),
        scratch_types=[
            pltpu.VMEM((L,), jnp.float32),
            pltpu.VMEM((16,), jnp.int32),
            pltpu.VMEM((16,), jnp.float32),
            pltpu.VMEM((16, D), jnp.float32),
            pltpu.SemaphoreType.DMA,
        ],
    )(ltT, x2d)


def _c_body(ws_ref, rows_ref, tv_ref, Ww_ref, bw_ref, Wk_ref, bk_ref,
            Wv_ref, bv_ref, gpost_ref, hmask_ref, wsn_ref, kk_ref, vv_ref):
    hmask = hmask_ref[...]
    for b in range(B):
        tv = tv_ref[b][:, :K]
        mx = jnp.max(tv, axis=-1, keepdims=True)
        e = jnp.exp(tv - mx)
        w = e / jnp.sum(e, axis=-1, keepdims=True)
        rows = rows_ref[b].reshape(S * K, D)
        y = lax.dot_general(rows, Ww_ref[...], (((1,), (1,)), ((), ())),
                            preferred_element_type=jnp.float32)
        y = (y + bw_ref[...][None, :]).reshape(S, K, D)
        wb = w.astype(jnp.bfloat16).astype(jnp.float32)
        yb = y.astype(jnp.bfloat16).astype(jnp.float32)
        written = wb[:, 0:1] * yb[:, 0, :]
        for j in range(1, K):
            written = written + wb[:, j:j + 1] * yb[:, j, :]
        t = ws_ref[b] + written
        wsn = _rms(t, gpost_ref[...])
        wsn_ref[b] = wsn
        kk = lax.dot_general(wsn, Wk_ref[...], (((1,), (1,)), ((), ())),
                             preferred_element_type=jnp.float32) \
            + bk_ref[...][None, :]
        vv = lax.dot_general(wsn, Wv_ref[...], (((1,), (1,)), ((), ())),
                             preferred_element_type=jnp.float32) \
            + bv_ref[...][None, :]
        kk_ref[b] = jnp.broadcast_to(kk[None], (H, S, D)).reshape(HS, D) * hmask
        vv_ref[b] = jnp.broadcast_to(vv[None], (H, S, D)).reshape(HS, D) * hmask


def _call_c(ws, rows, tv, Ww, bw, Wk, bk, Wv, bv, g_post, hmask):
    full = lambda shape: pl.BlockSpec(shape, lambda: tuple(0 for _ in shape))
    return pl.pallas_call(
        _c_body,
        in_specs=[full((B, S, D)), full((B, S, K, D)), full((B, S, 16)),
                  full((D, D)), full((D,)), full((D, D)), full((D,)),
                  full((D, D)), full((D,)), full((D,)), full((HS, D))],
        out_specs=[full((B, S, D)), full((B, HS, D)), full((B, HS, D))],
        out_shape=[jax.ShapeDtypeStruct((B, S, D), jnp.float32),
                   jax.ShapeDtypeStruct((B, HS, D), jnp.float32),
                   jax.ShapeDtypeStruct((B, HS, D), jnp.float32)],
    )(ws, rows, tv, Ww, bw, Wk, bk, Wv, bv, g_post, hmask)


def _d_body(x_ref, kk_ref, vv_ref, Wq_ref, bq_ref, Wo_ref, bo_ref,
            gpost_ref, wi_ref, bi_ref, *rest, is_last):
    if is_last:
        ig1_ref, xo_ref, ig_ref = rest
    else:
        xo_ref, ig_ref = rest
    inv_sqrt_dh = 1.0 / (DH ** 0.5)
    for b in range(B):
        xb = x_ref[b]
        q = lax.dot_general(xb, Wq_ref[...], (((1,), (1,)), ((), ())),
                            preferred_element_type=jnp.float32) \
            + bq_ref[...][None, :]
        sc = lax.dot_general(q, kk_ref[b], (((1,), (1,)), ((), ())),
                             preferred_element_type=jnp.float32)
        sc = sc * inv_sqrt_dh
        parts = []
        for h in range(H):
            g = sc[:, h * S:(h + 1) * S]
            mx = jnp.max(g, axis=-1, keepdims=True)
            eg = jnp.exp(g - mx)
            parts.append(eg / jnp.sum(eg, axis=-1, keepdims=True))
        att = jnp.concatenate(parts, axis=1)
        attv = lax.dot_general(att, vv_ref[b], (((1,), (0,)), ((), ())),
                               preferred_element_type=jnp.float32)
        xb2 = lax.dot_general(attv, Wo_ref[...], (((1,), (1,)), ((), ())),
                              preferred_element_type=jnp.float32) \
            + bo_ref[...][None, :]
        xn = _rms(xb + xb2, gpost_ref[...])
        xo_ref[b] = xn
        gl = lax.dot_general(xn, wi_ref[...][None, :], (((1,), (1,)), ((), ())),
                             preferred_element_type=jnp.float32)
        gate = jax.nn.sigmoid(gl[:, 0] + bi_ref[0])
        if is_last:
            ig_ref[b] = (ig1_ref[b] + gate) * 0.5
        else:
            ig_ref[b] = gate


def _call_d(x, kk, vv, Wq, bq, Wo, bo, g_post, wi, bi, ig_prev):
    is_last = ig_prev is not None
    in_specs = [
        pl.BlockSpec((B, TL, D), lambda t: (0, t, 0)),
        pl.BlockSpec((B, HS, D), lambda t: (0, 0, 0)),
        pl.BlockSpec((B, HS, D), lambda t: (0, 0, 0)),
        pl.BlockSpec((D, D), lambda t: (0, 0)),
        pl.BlockSpec((D,), lambda t: (0,)),
        pl.BlockSpec((D, D), lambda t: (0, 0)),
        pl.BlockSpec((D,), lambda t: (0,)),
        pl.BlockSpec((D,), lambda t: (0,)),
        pl.BlockSpec((D,), lambda t: (0,)),
        pl.BlockSpec((1,), lambda t: (0,)),
    ]
    args = [x, kk, vv, Wq, bq, Wo, bo, g_post, wi, bi]
    if is_last:
        in_specs.append(pl.BlockSpec((B, TL), lambda t: (0, t)))
        args.append(ig_prev)
    return pl.pallas_call(
        functools.partial(_d_body, is_last=is_last),
        grid=(L // TL,),
        in_specs=in_specs,
        out_specs=[pl.BlockSpec((B, TL, D), lambda t: (0, t, 0)),
                   pl.BlockSpec((B, TL), lambda t: (0, t))],
        out_shape=[jax.ShapeDtypeStruct((B, L, D), jnp.float32),
                   jax.ShapeDtypeStruct((B, L), jnp.float32)],
    )(*args)


def kernel(x, workspace, Wc, bc, Ww, bw, Wq, bq, Wk, bk, Wv, bv, Wo, bo,
           Wi, bi, g_pre, g_post):
    wi = Wi[0]
    hmask = jnp.kron(jnp.eye(H, dtype=jnp.float32),
                     jnp.ones((S, D // H), jnp.float32))
    ws = jnp.broadcast_to(workspace[None], (B, S, D))
    ig = None
    for _ in range(NB):
        ltT = _call_a(x, Wc, bc, g_pre)
        _, tv, rows = _sc_topk(ltT, x.reshape(B * L, D))
        ws, kk, vv = _call_c(ws, rows, tv, Ww, bw, Wk, bk, Wv, bv, g_post,
                             hmask)
        x, ig = _call_d(x, kk, vv, Wq, bq, Wo, bo, g_post, wi, bi, ig)
    return x, ws, ig


# speedup vs baseline: 1.8768x; 1.8768x over previous
import functools

import jax
import jax.numpy as jnp
from jax import lax
from jax.experimental import pallas as pl
from jax.experimental.pallas import tpu as pltpu
from jax.experimental.pallas import tpu_sc as plsc

B, L, D, S, K, H, NB = 2, 2048, 1024, 16, 4, 16, 2
DH = D // H
HS = H * S
TL = 512
EPS = 1e-6
NEG = -3.0e38


def _rms(t, g):
    return t * lax.rsqrt(jnp.mean(t * t, axis=-1, keepdims=True) + EPS) * g[None, :]


def _a_body(x_ref, Wc_ref, bc_ref, gpre_ref, lt_ref):
    for b in range(B):
        xn = _rms(x_ref[b], gpre_ref[...])
        lgT = lax.dot_general(Wc_ref[...], xn, (((1,), (1,)), ((), ())),
                              preferred_element_type=jnp.float32)
        lt_ref[b] = lgT + bc_ref[...][:, None]


def _call_a(x, Wc, bc, g_pre):
    return pl.pallas_call(
        _a_body,
        grid=(L // TL,),
        in_specs=[
            pl.BlockSpec((B, TL, D), lambda t: (0, t, 0)),
            pl.BlockSpec((S, D), lambda t: (0, 0)),
            pl.BlockSpec((S,), lambda t: (0,)),
            pl.BlockSpec((D,), lambda t: (0,)),
        ],
        out_specs=pl.BlockSpec((B, S, TL), lambda t: (0, 0, t)),
        out_shape=jax.ShapeDtypeStruct((B, S, L), jnp.float32),
    )(x, Wc, bc, g_pre)


def _b_body(lt_hbm, x2d_hbm, ti_hbm, tv_hbm, rows_hbm,
            lg_v, iv_v, tv_v, rows_v, sem):
    wid = lax.axis_index("c") * 16 + lax.axis_index("s")
    b = wid // S
    s = wid % S
    pltpu.sync_copy(lt_hbm.at[b, s], lg_v)
    lane = lax.iota(jnp.int32, 16)

    UNROLL = 8
    selv = []
    seli = []
    for _ in range(K):
        prev = list(seli)

        def scan_body(i, carry, _prev=prev):
            bv, bi_ = carry
            cand = []
            for u in range(UNROLL):
                off = pl.multiple_of((i * UNROLL + u) * 16, 16)
                v = lg_v[pl.ds(off, 16)]
                idx = off + lane
                for pj in _prev:
                    v = jnp.where(idx == pj, NEG, v)
                cand.append((v, idx))
            while len(cand) > 1:
                nxt = []
                for j in range(0, len(cand), 2):
                    (va, ia), (vb, ib) = cand[j], cand[j + 1]
                    keep_a = va >= vb
                    nxt.append((jnp.where(keep_a, va, vb),
                                jnp.where(keep_a, ia, ib)))
                cand = nxt
            v, idx = cand[0]
            upd = v > bv
            return jnp.where(upd, v, bv), jnp.where(upd, idx, bi_)

        bv, bi_ = lax.fori_loop(
            0, L // (16 * UNROLL), scan_body,
            (jnp.full((16,), NEG, jnp.float32), jnp.zeros((16,), jnp.int32)))
        m = jnp.max(bv)
        cand = jnp.where(bv == m, bi_, jnp.int32(2 ** 30))
        selv.append(m)
        seli.append(jnp.min(cand))

    for a, c in ((0, 1), (2, 3), (0, 2), (1, 3), (1, 2)):
        swap = seli[a] > seli[c]
        ia, ic = seli[a], seli[c]
        va, vc = selv[a], selv[c]
        seli[a] = jnp.where(swap, ic, ia)
        seli[c] = jnp.where(swap, ia, ic)
        selv[a] = jnp.where(swap, vc, va)
        selv[c] = jnp.where(swap, va, vc)

    vvec = jnp.zeros((16,), jnp.float32)
    ivec = jnp.zeros((16,), jnp.int32)
    for j in range(K):
        vvec = jnp.where(lane == j, selv[j], vvec)
        ivec = jnp.where(lane == j, seli[j], ivec)
    iv_v[...] = ivec
    tv_v[...] = vvec
    pltpu.sync_copy(iv_v, ti_hbm.at[b, s])
    pltpu.sync_copy(tv_v, tv_hbm.at[b, s])

    gidx = ivec + b * L
    pltpu.async_copy(x2d_hbm.at[gidx], rows_v, sem).wait()
    pltpu.sync_copy(rows_v.at[pl.ds(0, K)], rows_hbm.at[b, s])


def _sc_topk(ltT, x2d):
    mesh = plsc.VectorSubcoreMesh(core_axis_name="c", subcore_axis_name="s")
    return pl.kernel(
        _b_body,
        out_type=[jax.ShapeDtypeStruct((B, S, 16), jnp.int32),
                  jax.ShapeDtypeStruct((B, S, 16), jnp.float32),
                  jax.ShapeDtypeStruct((B, S, K, D), jnp.float32)],
        mesh=mesh,
        compiler_params=pltpu.CompilerParams(needs_layout_passes=False),
        scratch_types=[
            pltpu.VMEM((L,), jnp.float32),
            pltpu.VMEM((16,), jnp.int32),
            pltpu.VMEM((16,), jnp.float32),
            pltpu.VMEM((16, D), jnp.float32),
            pltpu.SemaphoreType.DMA,
        ],
    )(ltT, x2d)


def _c_body(ws_ref, rows_ref, tv_ref, Ww_ref, bw_ref, Wk_ref, bk_ref,
            Wv_ref, bv_ref, gpost_ref, wsn_ref, kk_ref, vv_ref):
    for b in range(B):
        tv = tv_ref[b][:, :K]
        mx = jnp.max(tv, axis=-1, keepdims=True)
        e = jnp.exp(tv - mx)
        w = e / jnp.sum(e, axis=-1, keepdims=True)
        rows = rows_ref[b].reshape(S * K, D)
        y = lax.dot_general(rows, Ww_ref[...], (((1,), (1,)), ((), ())),
                            preferred_element_type=jnp.float32)
        y = (y + bw_ref[...][None, :]).reshape(S, K, D)
        wb = w.astype(jnp.bfloat16).astype(jnp.float32)
        yb = y.astype(jnp.bfloat16).astype(jnp.float32)
        written = wb[:, 0:1] * yb[:, 0, :]
        for j in range(1, K):
            written = written + wb[:, j:j + 1] * yb[:, j, :]
        t = ws_ref[b] + written
        wsn = _rms(t, gpost_ref[...])
        wsn_ref[b] = wsn
        kk = lax.dot_general(wsn, Wk_ref[...], (((1,), (1,)), ((), ())),
                             preferred_element_type=jnp.float32) \
            + bk_ref[...][None, :]
        kk_ref[b] = lax.dot_general(wsn, Wk_ref[...], (((1,), (1,)), ((), ())),
                                    preferred_element_type=jnp.float32) \
            + bk_ref[...][None, :]
        vv_ref[b] = lax.dot_general(wsn, Wv_ref[...], (((1,), (1,)), ((), ())),
                                    preferred_element_type=jnp.float32) \
            + bv_ref[...][None, :]


def _call_c(ws, rows, tv, Ww, bw, Wk, bk, Wv, bv, g_post):
    full = lambda shape: pl.BlockSpec(shape, lambda: tuple(0 for _ in shape))
    return pl.pallas_call(
        _c_body,
        in_specs=[full((B, S, D)), full((B, S, K, D)), full((B, S, 16)),
                  full((D, D)), full((D,)), full((D, D)), full((D,)),
                  full((D, D)), full((D,)), full((D,))],
        out_specs=[full((B, S, D)), full((B, S, D)), full((B, S, D))],
        out_shape=[jax.ShapeDtypeStruct((B, S, D), jnp.float32)] * 3,
    )(ws, rows, tv, Ww, bw, Wk, bk, Wv, bv, g_post)


def _d_body(x_ref, kk_ref, vv_ref, Wq_ref, bq_ref, Wo_ref, bo_ref,
            gpost_ref, wi_ref, bi_ref, *rest, is_last):
    if is_last:
        ig1_ref, xo_ref, ig_ref = rest
    else:
        xo_ref, ig_ref = rest
    inv_sqrt_dh = 1.0 / (DH ** 0.5)
    for b in range(B):
        xb = x_ref[b]
        q = lax.dot_general(xb, Wq_ref[...], (((1,), (1,)), ((), ())),
                            preferred_element_type=jnp.float32) \
            + bq_ref[...][None, :]
        kk = kk_ref[b]
        vv = vv_ref[b]
        parts = []
        for h in range(H):
            sl = slice(h * DH, (h + 1) * DH)
            scT = lax.dot_general(kk[:, sl], q[:, sl], (((1,), (1,)), ((), ())),
                                  preferred_element_type=jnp.float32)
            scT = scT * inv_sqrt_dh
            mx = jnp.max(scT, axis=0, keepdims=True)
            eg = jnp.exp(scT - mx)
            attT = eg / jnp.sum(eg, axis=0, keepdims=True)
            parts.append(lax.dot_general(attT, vv[:, sl], (((0,), (0,)), ((), ())),
                                         preferred_element_type=jnp.float32))
        attv = jnp.concatenate(parts, axis=1)
        xb2 = lax.dot_general(attv, Wo_ref[...], (((1,), (1,)), ((), ())),
                              preferred_element_type=jnp.float32) \
            + bo_ref[...][None, :]
        xn = _rms(xb + xb2, gpost_ref[...])
        xo_ref[b] = xn
        gl = lax.dot_general(xn, wi_ref[...][None, :], (((1,), (1,)), ((), ())),
                             preferred_element_type=jnp.float32)
        gate = jax.nn.sigmoid(gl[:, 0] + bi_ref[0])
        if is_last:
            ig_ref[b] = (ig1_ref[b] + gate) * 0.5
        else:
            ig_ref[b] = gate


def _call_d(x, kk, vv, Wq, bq, Wo, bo, g_post, wi, bi, ig_prev):
    is_last = ig_prev is not None
    in_specs = [
        pl.BlockSpec((B, TL, D), lambda t: (0, t, 0)),
        pl.BlockSpec((B, S, D), lambda t: (0, 0, 0)),
        pl.BlockSpec((B, S, D), lambda t: (0, 0, 0)),
        pl.BlockSpec((D, D), lambda t: (0, 0)),
        pl.BlockSpec((D,), lambda t: (0,)),
        pl.BlockSpec((D, D), lambda t: (0, 0)),
        pl.BlockSpec((D,), lambda t: (0,)),
        pl.BlockSpec((D,), lambda t: (0,)),
        pl.BlockSpec((D,), lambda t: (0,)),
        pl.BlockSpec((1,), lambda t: (0,)),
    ]
    args = [x, kk, vv, Wq, bq, Wo, bo, g_post, wi, bi]
    if is_last:
        in_specs.append(pl.BlockSpec((B, TL), lambda t: (0, t)))
        args.append(ig_prev)
    return pl.pallas_call(
        functools.partial(_d_body, is_last=is_last),
        grid=(L // TL,),
        in_specs=in_specs,
        out_specs=[pl.BlockSpec((B, TL, D), lambda t: (0, t, 0)),
                   pl.BlockSpec((B, TL), lambda t: (0, t))],
        out_shape=[jax.ShapeDtypeStruct((B, L, D), jnp.float32),
                   jax.ShapeDtypeStruct((B, L), jnp.float32)],
    )(*args)


def kernel(x, workspace, Wc, bc, Ww, bw, Wq, bq, Wk, bk, Wv, bv, Wo, bo,
           Wi, bi, g_pre, g_post):
    wi = Wi[0]
    ws = jnp.broadcast_to(workspace[None], (B, S, D))
    ig = None
    for _ in range(NB):
        ltT = _call_a(x, Wc, bc, g_pre)
        _, tv, rows = _sc_topk(ltT, x.reshape(B * L, D))
        ws, kk, vv = _call_c(ws, rows, tv, Ww, bw, Wk, bk, Wv, bv, g_post)
        x, ig = _call_d(x, kk, vv, Wq, bq, Wo, bo, g_post, wi, bi, ig)
    return x, ws, ig


# speedup vs baseline: 1.9318x; 1.0293x over previous
import functools

import jax
import jax.numpy as jnp
from jax import lax
from jax.experimental import pallas as pl
from jax.experimental.pallas import tpu as pltpu
from jax.experimental.pallas import tpu_sc as plsc

B, L, D, S, K, H, NB = 2, 2048, 1024, 16, 4, 16, 2
DH = D // H
HS = H * S
TL = 512
EPS = 1e-6
NEG = -3.0e38


def _rms(t, g):
    return t * lax.rsqrt(jnp.mean(t * t, axis=-1, keepdims=True) + EPS) * g[None, :]


def _a_body(x_ref, Wc_ref, bc_ref, gpre_ref, lt_ref):
    for b in range(B):
        xn = _rms(x_ref[b], gpre_ref[...])
        lgT = lax.dot_general(Wc_ref[...], xn, (((1,), (1,)), ((), ())),
                              preferred_element_type=jnp.float32)
        lt_ref[b] = lgT + bc_ref[...][:, None]


def _call_a(x, Wc, bc, g_pre):
    return pl.pallas_call(
        _a_body,
        grid=(L // TL,),
        in_specs=[
            pl.BlockSpec((B, TL, D), lambda t: (0, t, 0)),
            pl.BlockSpec((S, D), lambda t: (0, 0)),
            pl.BlockSpec((S,), lambda t: (0,)),
            pl.BlockSpec((D,), lambda t: (0,)),
        ],
        out_specs=pl.BlockSpec((B, S, TL), lambda t: (0, 0, t)),
        out_shape=jax.ShapeDtypeStruct((B, S, L), jnp.float32),
    )(x, Wc, bc, g_pre)


def _b_body(lt_hbm, x2d_hbm, ti_hbm, tv_hbm, rows_hbm,
            lg_v, iv_v, tv_v, rows_v, sem):
    wid = lax.axis_index("c") * 16 + lax.axis_index("s")
    b = wid // S
    s = wid % S
    pltpu.sync_copy(lt_hbm.at[b, s], lg_v)
    lane = lax.iota(jnp.int32, 16)

    UNROLL = 8
    selv = []
    seli = []
    for _ in range(K):
        prev = list(seli)

        def scan_body(i, carry, _prev=prev):
            bv, bi_ = carry
            cand = []
            for u in range(UNROLL):
                off = pl.multiple_of((i * UNROLL + u) * 16, 16)
                v = lg_v[pl.ds(off, 16)]
                idx = off + lane
                for pj in _prev:
                    v = jnp.where(idx == pj, NEG, v)
                cand.append((v, idx))
            while len(cand) > 1:
                nxt = []
                for j in range(0, len(cand), 2):
                    (va, ia), (vb, ib) = cand[j], cand[j + 1]
                    keep_a = va >= vb
                    nxt.append((jnp.where(keep_a, va, vb),
                                jnp.where(keep_a, ia, ib)))
                cand = nxt
            v, idx = cand[0]
            upd = v > bv
            return jnp.where(upd, v, bv), jnp.where(upd, idx, bi_)

        bv, bi_ = lax.fori_loop(
            0, L // (16 * UNROLL), scan_body,
            (jnp.full((16,), NEG, jnp.float32), jnp.zeros((16,), jnp.int32)))
        m = jnp.max(bv)
        cand = jnp.where(bv == m, bi_, jnp.int32(2 ** 30))
        selv.append(m)
        seli.append(jnp.min(cand))

    for a, c in ((0, 1), (2, 3), (0, 2), (1, 3), (1, 2)):
        swap = seli[a] > seli[c]
        ia, ic = seli[a], seli[c]
        va, vc = selv[a], selv[c]
        seli[a] = jnp.where(swap, ic, ia)
        seli[c] = jnp.where(swap, ia, ic)
        selv[a] = jnp.where(swap, vc, va)
        selv[c] = jnp.where(swap, va, vc)

    vvec = jnp.zeros((16,), jnp.float32)
    ivec = jnp.zeros((16,), jnp.int32)
    for j in range(K):
        vvec = jnp.where(lane == j, selv[j], vvec)
        ivec = jnp.where(lane == j, seli[j], ivec)
    iv_v[...] = ivec
    tv_v[...] = vvec
    pltpu.sync_copy(iv_v, ti_hbm.at[b, s])
    pltpu.sync_copy(tv_v, tv_hbm.at[b, s])

    gidx = ivec + b * L
    pltpu.async_copy(x2d_hbm.at[gidx], rows_v, sem).wait()
    pltpu.sync_copy(rows_v.at[pl.ds(0, K)], rows_hbm.at[b, s])


def _sc_topk(ltT, x2d):
    mesh = plsc.VectorSubcoreMesh(core_axis_name="c", subcore_axis_name="s")
    return pl.kernel(
        _b_body,
        out_type=[jax.ShapeDtypeStruct((B, S, 16), jnp.int32),
                  jax.ShapeDtypeStruct((B, S, 16), jnp.float32),
                  jax.ShapeDtypeStruct((B, S, K, D), jnp.float32)],
        mesh=mesh,
        compiler_params=pltpu.CompilerParams(needs_layout_passes=False),
        scratch_types=[
            pltpu.VMEM((L,), jnp.float32),
            pltpu.VMEM((16,), jnp.int32),
            pltpu.VMEM((16,), jnp.float32),
            pltpu.VMEM((16, D), jnp.float32),
            pltpu.SemaphoreType.DMA,
        ],
    )(ltT, x2d)


def _c_body(ws_ref, rows_ref, tv_ref, Ww_ref, bw_ref, Wk_ref, bk_ref,
            Wv_ref, bv_ref, gpost_ref, wsn_ref, kk_ref, vv_ref):
    for b in range(B):
        tv = tv_ref[b][:, :K]
        mx = jnp.max(tv, axis=-1, keepdims=True)
        e = jnp.exp(tv - mx)
        w = e / jnp.sum(e, axis=-1, keepdims=True)
        rows = rows_ref[b].reshape(S * K, D)
        y = lax.dot_general(rows, Ww_ref[...], (((1,), (1,)), ((), ())),
                            preferred_element_type=jnp.float32)
        y = (y + bw_ref[...][None, :]).reshape(S, K, D)
        wb = w.astype(jnp.bfloat16).astype(jnp.float32)
        yb = y.astype(jnp.bfloat16).astype(jnp.float32)
        written = wb[:, 0:1] * yb[:, 0, :]
        for j in range(1, K):
            written = written + wb[:, j:j + 1] * yb[:, j, :]
        t = ws_ref[b] + written
        wsn = _rms(t, gpost_ref[...])
        wsn_ref[b] = wsn
        kk = lax.dot_general(wsn, Wk_ref[...], (((1,), (1,)), ((), ())),
                             preferred_element_type=jnp.float32) \
            + bk_ref[...][None, :]
        kk_ref[b] = lax.dot_general(wsn, Wk_ref[...], (((1,), (1,)), ((), ())),
                                    preferred_element_type=jnp.float32) \
            + bk_ref[...][None, :]
        vv_ref[b] = lax.dot_general(wsn, Wv_ref[...], (((1,), (1,)), ((), ())),
                                    preferred_element_type=jnp.float32) \
            + bv_ref[...][None, :]


def _call_c(ws, rows, tv, Ww, bw, Wk, bk, Wv, bv, g_post):
    full = lambda shape: pl.BlockSpec(shape, lambda: tuple(0 for _ in shape))
    return pl.pallas_call(
        _c_body,
        in_specs=[full((B, S, D)), full((B, S, K, D)), full((B, S, 16)),
                  full((D, D)), full((D,)), full((D, D)), full((D,)),
                  full((D, D)), full((D,)), full((D,))],
        out_specs=[full((B, S, D)), full((B, S, D)), full((B, S, D))],
        out_shape=[jax.ShapeDtypeStruct((B, S, D), jnp.float32)] * 3,
    )(ws, rows, tv, Ww, bw, Wk, bk, Wv, bv, g_post)


def _d_body(x_ref, kk_ref, vv_ref, Wq_ref, bq_ref, Wo_ref, bo_ref,
            gpost_ref, wi_ref, bi_ref, Wc_ref, bc_ref, gpre_ref,
            *rest, is_last):
    if is_last:
        ig1_ref, xo_ref, ig_ref = rest
    else:
        xo_ref, ig_ref, lt_ref = rest
    inv_sqrt_dh = 1.0 / (DH ** 0.5)
    for b in range(B):
        xb = x_ref[b]
        q = lax.dot_general(xb, Wq_ref[...], (((1,), (1,)), ((), ())),
                            preferred_element_type=jnp.float32) \
            + bq_ref[...][None, :]
        kk = kk_ref[b]
        vv = vv_ref[b]
        parts = []
        for h in range(H):
            sl = slice(h * DH, (h + 1) * DH)
            scT = lax.dot_general(kk[:, sl], q[:, sl], (((1,), (1,)), ((), ())),
                                  preferred_element_type=jnp.float32)
            scT = scT * inv_sqrt_dh
            mx = jnp.max(scT, axis=0, keepdims=True)
            eg = jnp.exp(scT - mx)
            attT = eg / jnp.sum(eg, axis=0, keepdims=True)
            parts.append(lax.dot_general(attT, vv[:, sl], (((0,), (0,)), ((), ())),
                                         preferred_element_type=jnp.float32))
        attv = jnp.concatenate(parts, axis=1)
        xb2 = lax.dot_general(attv, Wo_ref[...], (((1,), (1,)), ((), ())),
                              preferred_element_type=jnp.float32) \
            + bo_ref[...][None, :]
        xn = _rms(xb + xb2, gpost_ref[...])
        xo_ref[b] = xn
        gl = lax.dot_general(xn, wi_ref[...][None, :], (((1,), (1,)), ((), ())),
                             preferred_element_type=jnp.float32)
        gate = jax.nn.sigmoid(gl[:, 0] + bi_ref[0])
        if is_last:
            ig_ref[b] = (ig1_ref[b] + gate) * 0.5
        else:
            ig_ref[b] = gate
            x2n = _rms(xn, gpre_ref[...])
            lgT = lax.dot_general(Wc_ref[...], x2n, (((1,), (1,)), ((), ())),
                                  preferred_element_type=jnp.float32)
            lt_ref[b] = lgT + bc_ref[...][:, None]


def _call_d(x, kk, vv, Wq, bq, Wo, bo, g_post, wi, bi, Wc, bc, g_pre,
            ig_prev):
    is_last = ig_prev is not None
    in_specs = [
        pl.BlockSpec((B, TL, D), lambda t: (0, t, 0)),
        pl.BlockSpec((B, S, D), lambda t: (0, 0, 0)),
        pl.BlockSpec((B, S, D), lambda t: (0, 0, 0)),
        pl.BlockSpec((D, D), lambda t: (0, 0)),
        pl.BlockSpec((D,), lambda t: (0,)),
        pl.BlockSpec((D, D), lambda t: (0, 0)),
        pl.BlockSpec((D,), lambda t: (0,)),
        pl.BlockSpec((D,), lambda t: (0,)),
        pl.BlockSpec((D,), lambda t: (0,)),
        pl.BlockSpec((1,), lambda t: (0,)),
        pl.BlockSpec((S, D), lambda t: (0, 0)),
        pl.BlockSpec((S,), lambda t: (0,)),
        pl.BlockSpec((D,), lambda t: (0,)),
    ]
    args = [x, kk, vv, Wq, bq, Wo, bo, g_post, wi, bi, Wc, bc, g_pre]
    out_specs = [pl.BlockSpec((B, TL, D), lambda t: (0, t, 0)),
                 pl.BlockSpec((B, TL), lambda t: (0, t))]
    out_shape = [jax.ShapeDtypeStruct((B, L, D), jnp.float32),
                 jax.ShapeDtypeStruct((B, L), jnp.float32)]
    if is_last:
        in_specs.append(pl.BlockSpec((B, TL), lambda t: (0, t)))
        args.append(ig_prev)
    else:
        out_specs.append(pl.BlockSpec((B, S, TL), lambda t: (0, 0, t)))
        out_shape.append(jax.ShapeDtypeStruct((B, S, L), jnp.float32))
    return pl.pallas_call(
        functools.partial(_d_body, is_last=is_last),
        grid=(L // TL,),
        in_specs=in_specs,
        out_specs=out_specs,
        out_shape=out_shape,
    )(*args)


def kernel(x, workspace, Wc, bc, Ww, bw, Wq, bq, Wk, bk, Wv, bv, Wo, bo,
           Wi, bi, g_pre, g_post):
    wi = Wi[0]
    ws = jnp.broadcast_to(workspace[None], (B, S, D))
    ig = None
    ltT = _call_a(x, Wc, bc, g_pre)
    for blk in range(NB):
        _, tv, rows = _sc_topk(ltT, x.reshape(B * L, D))
        ws, kk, vv = _call_c(ws, rows, tv, Ww, bw, Wk, bk, Wv, bv, g_post)
        out = _call_d(x, kk, vv, Wq, bq, Wo, bo, g_post, wi, bi,
                      Wc, bc, g_pre, ig)
        if blk + 1 < NB:
            x, ig, ltT = out
        else:
            x, ig = out
    return x, ws, ig
